# baseline (speedup 1.0000x reference)
import functools

import jax
import jax.numpy as jnp
from jax import lax
from jax.experimental import pallas as pl
from jax.experimental.pallas import tpu as pltpu
from jax.experimental.pallas import tpu_sc as plsc

N = 10000
NP = 10240
E = 320000
NC, NS, L = 2, 16, 16
NW = NC * NS
C = 125
CPT = E // (NW * C)
SB = 16
NSB = CPT // SB
RPT = NP // NS
ZR = 16

_MESH = plsc.VectorSubcoreMesh(
    core_axis_name="c", subcore_axis_name="s", num_cores=NC, num_subcores=NS
)
_SC_PARAMS = pltpu.CompilerParams(needs_layout_passes=False)


@functools.partial(
    pl.kernel,
    out_type=jax.ShapeDtypeStruct((NC, NP), jnp.float32),
    mesh=_MESH,
    scratch_types=[
        pltpu.VMEM((SB, C), jnp.int32),
        pltpu.VMEM((SB, C), jnp.float32),
        pltpu.VMEM((RPT,), jnp.float32),
        pltpu.VMEM_SHARED((NP,), jnp.float32),
    ],
    compiler_params=pltpu.CompilerParams(
        needs_layout_passes=False, use_tc_tiling_on_sc=False
    ),
)
def _deg_kernel(dst_hbm, w_hbm, out_hbm, dstb, wb, zbuf, acc):
    c = lax.axis_index("c")
    s = lax.axis_index("s")
    wid = s * NC + c
    zero = jnp.zeros((L,), jnp.float32)

    @pl.loop(0, RPT // L)
    def _(i):
        zbuf[pl.ds(i * L, L)] = zero

    row0 = s * RPT
    pltpu.sync_copy(zbuf, acc.at[pl.ds(row0, RPT)])
    plsc.subcore_barrier()

    @pl.loop(0, NSB)
    def _(b):
        pltpu.sync_copy(dst_hbm.at[wid, pl.ds(b * SB, SB)], dstb)
        pltpu.sync_copy(w_hbm.at[wid, pl.ds(b * SB, SB)], wb)

        @pl.loop(0, SB)
        def _(j):
            pltpu.sync_copy(wb.at[j], acc.at[dstb.at[j]], add=True)

    plsc.subcore_barrier()
    pltpu.sync_copy(acc.at[pl.ds(row0, RPT)], out_hbm.at[c, pl.ds(row0, RPT)])


def _make_agg(D):
    @functools.partial(
        pl.kernel,
        out_type=jax.ShapeDtypeStruct((NC, NP, D), jnp.float32),
        mesh=_MESH,
        scratch_types=[
            pltpu.VMEM((SB, C), jnp.int32),
            pltpu.VMEM((SB, C), jnp.int32),
            pltpu.VMEM((SB, C), jnp.float32),
            pltpu.VMEM((C, D), jnp.float32),
            pltpu.VMEM((C, D), jnp.float32),
            pltpu.VMEM((ZR, D), jnp.float32),
            pltpu.VMEM_SHARED((NP, D), jnp.float32),
            pltpu.SemaphoreType.DMA,
            pltpu.SemaphoreType.DMA,
            pltpu.SemaphoreType.DMA,
            pltpu.SemaphoreType.DMA,
        ],
        compiler_params=_SC_PARAMS,
    )
    def _agg(h_hbm, src_hbm, dst_hbm, w_hbm, out_hbm,
             srcb, dstb, wb, rows0, rows1, zbuf, acc,
             gsem0, gsem1, ssem0, ssem1):
        c = lax.axis_index("c")
        s = lax.axis_index("s")
        wid = s * NC + c

        zero = jnp.zeros((L,), jnp.float32)

        @pl.loop(0, ZR)
        def _(i):
            for d in range(D // L):
                zbuf[i, pl.ds(d * L, L)] = zero

        row0 = s * RPT
        for k in range(RPT // ZR):
            pltpu.sync_copy(zbuf, acc.at[pl.ds(row0 + k * ZR, ZR)])
        plsc.subcore_barrier()

        rbufs = (rows0, rows1)
        gsems = (gsem0, gsem1)
        ssems = (ssem0, ssem1)

        @pl.loop(0, NSB)
        def _(b):
            pltpu.sync_copy(src_hbm.at[wid, pl.ds(b * SB, SB)], srcb)
            pltpu.sync_copy(dst_hbm.at[wid, pl.ds(b * SB, SB)], dstb)
            pltpu.sync_copy(w_hbm.at[wid, pl.ds(b * SB, SB)], wb)

            gd = {0: pltpu.async_copy(h_hbm.at[srcb.at[0]], rows0, gsem0)}
            sd = {}
            for jj in range(SB):
                p = jj % 2
                rb = rbufs[p]
                gd[jj].wait()
                if jj + 1 < SB:
                    if jj >= 1:
                        sd[jj - 1].wait()
                    q = (jj + 1) % 2
                    gd[jj + 1] = pltpu.async_copy(
                        h_hbm.at[srcb.at[jj + 1]], rbufs[q], gsems[q]
                    )

                @plsc.parallel_loop(0, C, unroll=10)
                def _(i, jj=jj, rb=rb):
                    j16 = jnp.full((L,), jj, jnp.int32)
                    i16 = jnp.full((L,), i, jnp.int32)
                    wv = plsc.load_gather(wb, [j16, i16])
                    for d in range(D // L):
                        rb[i, pl.ds(d * L, L)] = rb[i, pl.ds(d * L, L)] * wv

                sd[jj] = pltpu.async_copy(
                    rb, acc.at[dstb.at[jj]], ssems[p], add=True
                )
            sd[SB - 2].wait()
            sd[SB - 1].wait()

        plsc.subcore_barrier()
        pltpu.sync_copy(acc.at[pl.ds(row0, RPT)],
                        out_hbm.at[c, pl.ds(row0, RPT)])

    return _agg


_agg128 = _make_agg(128)


_BR = 1000


def _prep_body(x_ref, degp_ref, wc_ref, ws_ref, bs_ref,
               h1s_ref, hls_ref, hsp_ref, dinv_ref):
    xb = x_ref[...]
    deg = 1.0 + jnp.sum(degp_ref[...], axis=1)
    dinv = jnp.where(deg > 0, lax.rsqrt(deg), 0.0)
    h = jnp.dot(xb, wc_ref[...], preferred_element_type=jnp.float32)
    h = h * dinv[:, None]
    h1s_ref[...] = h[:, :128]
    hls_ref[...] = h[:, 128:]
    hsp_ref[...] = (
        jnp.dot(xb, ws_ref[...], preferred_element_type=jnp.float32)
        + bs_ref[...]
    )
    dinv_ref[...] = dinv[:, None]


def _mid_body(p1_ref, pl_ref, h1s_ref, hls_ref, dinv_ref, hsp_ref,
              b1_ref, bl1_ref, wl2_ref, bl2_ref, w2_ref,
              h2s_ref, leader_ref):
    dinv = dinv_ref[...]
    t1 = (p1_ref[0] + p1_ref[1] + h1s_ref[...]) * dinv
    tl = (pl_ref[0] + pl_ref[1] + hls_ref[...]) * dinv
    hn = jnp.maximum(t1 + b1_ref[...], 0.0)
    hl = jnp.maximum(tl[:, :64] + bl1_ref[...], 0.0)
    logit = jnp.sum(hl * wl2_ref[...], axis=1, keepdims=True) + bl2_ref[...]
    leader = jax.nn.sigmoid(logit)
    hnew = (1.0 - leader) * hn + leader * hsp_ref[...]
    h2s_ref[...] = (
        jnp.dot(hnew, w2_ref[...], preferred_element_type=jnp.float32) * dinv
    )
    leader_ref[...] = leader


def _fin_body(q_ref, h2s_ref, dinv_ref, b2_ref, out_ref):
    t = (q_ref[0] + q_ref[1] + h2s_ref[...]) * dinv_ref[...]
    out_ref[...] = jnp.maximum(t + b2_ref[...], 0.0)


def _row_spec(shape):
    if len(shape) == 2:
        return pl.BlockSpec((_BR, shape[1]), lambda i: (i, 0))
    return pl.BlockSpec((shape[0], _BR, shape[2]), lambda i: (0, i, 0))


def _full_spec(shape):
    return pl.BlockSpec(shape, lambda i: tuple(0 for _ in shape))


def kernel(x, edge_index, edge_weight, W1, b1, W2, b2, Ws, bs,
           Wl1, bl1, Wl2, bl2):
    src = edge_index[0].reshape(NW, CPT, C)
    dst = edge_index[1].reshape(NW, CPT, C)
    w = edge_weight.reshape(NW, CPT, C)

    degp = _deg_kernel(dst, w)[:, :N].T

    Wc = jnp.concatenate(
        [W1, Wl1, jnp.zeros((128, 64), jnp.float32)], axis=1
    )
    grid = (N // _BR,)
    h1s, hls, hsp, dinv = pl.pallas_call(
        _prep_body,
        grid=grid,
        in_specs=[
            _row_spec((N, 128)),
            pl.BlockSpec((_BR, NC), lambda i: (i, 0)),
            _full_spec((128, 256)),
            _full_spec((128, 128)),
            _full_spec((1, 128)),
        ],
        out_specs=[
            _row_spec((N, 128)),
            _row_spec((N, 128)),
            _row_spec((N, 128)),
            _row_spec((N, 1)),
        ],
        out_shape=[
            jax.ShapeDtypeStruct((N, 128), jnp.float32),
            jax.ShapeDtypeStruct((N, 128), jnp.float32),
            jax.ShapeDtypeStruct((N, 128), jnp.float32),
            jax.ShapeDtypeStruct((N, 1), jnp.float32),
        ],
    )(x, degp, Wc, Ws, bs.reshape(1, 128))

    p1 = _agg128(h1s, src, dst, w)
    plp = _agg128(hls, src, dst, w)

    h2s, leader = pl.pallas_call(
        _mid_body,
        grid=grid,
        in_specs=[
            _row_spec((NC, NP, 128)),
            _row_spec((NC, NP, 128)),
            _row_spec((N, 128)),
            _row_spec((N, 128)),
            _row_spec((N, 1)),
            _row_spec((N, 128)),
            _full_spec((1, 128)),
            _full_spec((1, 64)),
            _full_spec((1, 64)),
            _full_spec((1, 1)),
            _full_spec((128, 128)),
        ],
        out_specs=[
            _row_spec((N, 128)),
            _row_spec((N, 1)),
        ],
        out_shape=[
            jax.ShapeDtypeStruct((N, 128), jnp.float32),
            jax.ShapeDtypeStruct((N, 1), jnp.float32),
        ],
    )(p1, plp, h1s, hls, dinv, hsp, b1.reshape(1, 128), bl1.reshape(1, 64),
      Wl2.reshape(1, 64), bl2.reshape(1, 1), W2)

    q = _agg128(h2s, src, dst, w)

    h_final = pl.pallas_call(
        _fin_body,
        grid=grid,
        in_specs=[
            _row_spec((NC, NP, 128)),
            _row_spec((N, 128)),
            _row_spec((N, 1)),
            _full_spec((1, 128)),
        ],
        out_specs=_row_spec((N, 128)),
        out_shape=jax.ShapeDtypeStruct((N, 128), jnp.float32),
    )(q, h2s, dinv, b2.reshape(1, 128))

    return (h_final, leader)


# speedup vs baseline: 1.1318x; 1.1318x over previous
import functools

import jax
import jax.numpy as jnp
from jax import lax
from jax.experimental import pallas as pl
from jax.experimental.pallas import tpu as pltpu
from jax.experimental.pallas import tpu_sc as plsc

N = 10000
NP = 10240
E = 320000
NC, NS, L = 2, 16, 16
NW = NC * NS
C = 125
CPT = E // (NW * C)
SB = 16
NSB = CPT // SB
RPT = NP // NS
ZR = 16

_MESH = plsc.VectorSubcoreMesh(
    core_axis_name="c", subcore_axis_name="s", num_cores=NC, num_subcores=NS
)
_SC_PARAMS = pltpu.CompilerParams(needs_layout_passes=False)


@functools.partial(
    pl.kernel,
    out_type=jax.ShapeDtypeStruct((NC, NP), jnp.float32),
    mesh=_MESH,
    scratch_types=[
        pltpu.VMEM((SB, C), jnp.int32),
        pltpu.VMEM((SB, C), jnp.float32),
        pltpu.VMEM((RPT,), jnp.float32),
        pltpu.VMEM_SHARED((NP,), jnp.float32),
    ],
    compiler_params=pltpu.CompilerParams(
        needs_layout_passes=False, use_tc_tiling_on_sc=False
    ),
)
def _deg_kernel(dst_hbm, w_hbm, out_hbm, dstb, wb, zbuf, acc):
    c = lax.axis_index("c")
    s = lax.axis_index("s")
    wid = s * NC + c
    zero = jnp.zeros((L,), jnp.float32)

    @pl.loop(0, RPT // L)
    def _(i):
        zbuf[pl.ds(i * L, L)] = zero

    row0 = s * RPT
    pltpu.sync_copy(zbuf, acc.at[pl.ds(row0, RPT)])
    plsc.subcore_barrier()

    @pl.loop(0, NSB)
    def _(b):
        pltpu.sync_copy(dst_hbm.at[wid, pl.ds(b * SB, SB)], dstb)
        pltpu.sync_copy(w_hbm.at[wid, pl.ds(b * SB, SB)], wb)

        @pl.loop(0, SB)
        def _(j):
            pltpu.sync_copy(wb.at[j], acc.at[dstb.at[j]], add=True)

    plsc.subcore_barrier()
    pltpu.sync_copy(acc.at[pl.ds(row0, RPT)], out_hbm.at[c, pl.ds(row0, RPT)])


def _make_agg(D, tiled=True):
    @functools.partial(
        pl.kernel,
        out_type=jax.ShapeDtypeStruct((NC, NP, D), jnp.float32),
        mesh=_MESH,
        scratch_types=[
            pltpu.VMEM((SB, C), jnp.int32),
            pltpu.VMEM((SB, C), jnp.int32),
            pltpu.VMEM((SB, C), jnp.float32),
            pltpu.VMEM((C, D), jnp.float32),
            pltpu.VMEM((C, D), jnp.float32),
            pltpu.VMEM((ZR, D), jnp.float32),
            pltpu.VMEM_SHARED((NP, D), jnp.float32),
            pltpu.SemaphoreType.DMA,
            pltpu.SemaphoreType.DMA,
            pltpu.SemaphoreType.DMA,
            pltpu.SemaphoreType.DMA,
        ],
        compiler_params=(
            _SC_PARAMS
            if tiled
            else pltpu.CompilerParams(
                needs_layout_passes=False, use_tc_tiling_on_sc=False
            )
        ),
    )
    def _agg(h_hbm, src_hbm, dst_hbm, w_hbm, out_hbm,
             srcb, dstb, wb, rows0, rows1, zbuf, acc,
             gsem0, gsem1, ssem0, ssem1):
        c = lax.axis_index("c")
        s = lax.axis_index("s")
        wid = s * NC + c

        zero = jnp.zeros((L,), jnp.float32)

        @pl.loop(0, ZR)
        def _(i):
            for d in range(D // L):
                zbuf[i, pl.ds(d * L, L)] = zero

        row0 = s * RPT
        for k in range(RPT // ZR):
            pltpu.sync_copy(zbuf, acc.at[pl.ds(row0 + k * ZR, ZR)])
        plsc.subcore_barrier()

        rbufs = (rows0, rows1)
        gsems = (gsem0, gsem1)
        ssems = (ssem0, ssem1)

        @pl.loop(0, NSB)
        def _(b):
            pltpu.sync_copy(src_hbm.at[wid, pl.ds(b * SB, SB)], srcb)
            pltpu.sync_copy(dst_hbm.at[wid, pl.ds(b * SB, SB)], dstb)
            pltpu.sync_copy(w_hbm.at[wid, pl.ds(b * SB, SB)], wb)

            gd = {0: pltpu.async_copy(h_hbm.at[srcb.at[0]], rows0, gsem0)}
            sd = {}
            for jj in range(SB):
                p = jj % 2
                rb = rbufs[p]
                gd[jj].wait()
                if jj + 1 < SB:
                    if jj >= 1:
                        sd[jj - 1].wait()
                    q = (jj + 1) % 2
                    gd[jj + 1] = pltpu.async_copy(
                        h_hbm.at[srcb.at[jj + 1]], rbufs[q], gsems[q]
                    )

                @plsc.parallel_loop(0, C, unroll=5)
                def _(i, jj=jj, rb=rb):
                    j16 = jnp.full((L,), jj, jnp.int32)
                    i16 = jnp.full((L,), i, jnp.int32)
                    wv = plsc.load_gather(wb, [j16, i16])
                    for d in range(D // L):
                        rb[i, pl.ds(d * L, L)] = rb[i, pl.ds(d * L, L)] * wv

                sd[jj] = pltpu.async_copy(
                    rb, acc.at[dstb.at[jj]], ssems[p], add=True
                )
            sd[SB - 2].wait()
            sd[SB - 1].wait()

        plsc.subcore_barrier()
        pltpu.sync_copy(acc.at[pl.ds(row0, RPT)],
                        out_hbm.at[c, pl.ds(row0, RPT)])

    return _agg


_agg128 = _make_agg(128)
_agg64 = _make_agg(64, tiled=False)


_BR = 1000


def _prep_body(x_ref, degp_ref, wc_ref, ws_ref, bs_ref,
               h1s_ref, hls_ref, hsp_ref, dinv_ref):
    xb = x_ref[...]
    deg = 1.0 + jnp.sum(degp_ref[...], axis=1)
    dinv = jnp.where(deg > 0, lax.rsqrt(deg), 0.0)
    h = jnp.dot(xb, wc_ref[...], preferred_element_type=jnp.float32)
    h = h * dinv[:, None]
    h1s_ref[...] = h[:, :128]
    hls_ref[...] = h[:, 128:192]
    hsp_ref[...] = (
        jnp.dot(xb, ws_ref[...], preferred_element_type=jnp.float32)
        + bs_ref[...]
    )
    dinv_ref[...] = dinv[:, None]


def _mid_body(p1_ref, pl_ref, h1s_ref, hls_ref, dinv_ref, hsp_ref,
              b1_ref, bl1_ref, wl2_ref, bl2_ref, w2_ref,
              h2s_ref, leader_ref):
    dinv = dinv_ref[...]
    t1 = (p1_ref[0] + p1_ref[1] + h1s_ref[...]) * dinv
    tl = (pl_ref[0] + pl_ref[1] + hls_ref[...]) * dinv
    hn = jnp.maximum(t1 + b1_ref[...], 0.0)
    hl = jnp.maximum(tl + bl1_ref[...], 0.0)
    logit = jnp.sum(hl * wl2_ref[...], axis=1, keepdims=True) + bl2_ref[...]
    leader = jax.nn.sigmoid(logit)
    hnew = (1.0 - leader) * hn + leader * hsp_ref[...]
    h2s_ref[...] = (
        jnp.dot(hnew, w2_ref[...], preferred_element_type=jnp.float32) * dinv
    )
    leader_ref[...] = leader


def _fin_body(q_ref, h2s_ref, dinv_ref, b2_ref, out_ref):
    t = (q_ref[0] + q_ref[1] + h2s_ref[...]) * dinv_ref[...]
    out_ref[...] = jnp.maximum(t + b2_ref[...], 0.0)


def _row_spec(shape):
    if len(shape) == 2:
        return pl.BlockSpec((_BR, shape[1]), lambda i: (i, 0))
    return pl.BlockSpec((shape[0], _BR, shape[2]), lambda i: (0, i, 0))


def _full_spec(shape):
    return pl.BlockSpec(shape, lambda i: tuple(0 for _ in shape))


def kernel(x, edge_index, edge_weight, W1, b1, W2, b2, Ws, bs,
           Wl1, bl1, Wl2, bl2):
    src = edge_index[0].reshape(NW, CPT, C)
    dst = edge_index[1].reshape(NW, CPT, C)
    w = edge_weight.reshape(NW, CPT, C)

    degp = _deg_kernel(dst, w)[:, :N].T

    Wc = jnp.concatenate([W1, Wl1], axis=1)
    grid = (N // _BR,)
    h1s, hls, hsp, dinv = pl.pallas_call(
        _prep_body,
        grid=grid,
        in_specs=[
            _row_spec((N, 128)),
            pl.BlockSpec((_BR, NC), lambda i: (i, 0)),
            _full_spec((128, 192)),
            _full_spec((128, 128)),
            _full_spec((1, 128)),
        ],
        out_specs=[
            _row_spec((N, 128)),
            _row_spec((N, 64)),
            _row_spec((N, 128)),
            _row_spec((N, 1)),
        ],
        out_shape=[
            jax.ShapeDtypeStruct((N, 128), jnp.float32),
            jax.ShapeDtypeStruct((N, 64), jnp.float32),
            jax.ShapeDtypeStruct((N, 128), jnp.float32),
            jax.ShapeDtypeStruct((N, 1), jnp.float32),
        ],
    )(x, degp, Wc, Ws, bs.reshape(1, 128))

    p1 = _agg128(h1s, src, dst, w)
    plp = _agg64(hls, src, dst, w)

    h2s, leader = pl.pallas_call(
        _mid_body,
        grid=grid,
        in_specs=[
            _row_spec((NC, NP, 128)),
            _row_spec((NC, NP, 64)),
            _row_spec((N, 128)),
            _row_spec((N, 64)),
            _row_spec((N, 1)),
            _row_spec((N, 128)),
            _full_spec((1, 128)),
            _full_spec((1, 64)),
            _full_spec((1, 64)),
            _full_spec((1, 1)),
            _full_spec((128, 128)),
        ],
        out_specs=[
            _row_spec((N, 128)),
            _row_spec((N, 1)),
        ],
        out_shape=[
            jax.ShapeDtypeStruct((N, 128), jnp.float32),
            jax.ShapeDtypeStruct((N, 1), jnp.float32),
        ],
    )(p1, plp, h1s, hls, dinv, hsp, b1.reshape(1, 128), bl1.reshape(1, 64),
      Wl2.reshape(1, 64), bl2.reshape(1, 1), W2)

    q = _agg128(h2s, src, dst, w)

    h_final = pl.pallas_call(
        _fin_body,
        grid=grid,
        in_specs=[
            _row_spec((NC, NP, 128)),
            _row_spec((N, 128)),
            _row_spec((N, 1)),
            _full_spec((1, 128)),
        ],
        out_specs=_row_spec((N, 128)),
        out_shape=jax.ShapeDtypeStruct((N, 128), jnp.float32),
    )(q, h2s, dinv, b2.reshape(1, 128))

    return (h_final, leader)


# speedup vs baseline: 1.1445x; 1.0113x over previous
import functools

import jax
import jax.numpy as jnp
from jax import lax
from jax.experimental import pallas as pl
from jax.experimental.pallas import tpu as pltpu
from jax.experimental.pallas import tpu_sc as plsc

N = 10000
NP = 10240
E = 320000
NC, NS, L = 2, 16, 16
NW = NC * NS
C = 125
CPT = E // (NW * C)
SB = 40
NSB = CPT // SB
RPT = NP // NS
ZR = 8

_MESH = plsc.VectorSubcoreMesh(
    core_axis_name="c", subcore_axis_name="s", num_cores=NC, num_subcores=NS
)
_SC_PARAMS = pltpu.CompilerParams(needs_layout_passes=False)


@functools.partial(
    pl.kernel,
    out_type=jax.ShapeDtypeStruct((NC, NP), jnp.float32),
    mesh=_MESH,
    scratch_types=[
        pltpu.VMEM((SB, C), jnp.int32),
        pltpu.VMEM((SB, C), jnp.float32),
        pltpu.VMEM((RPT,), jnp.float32),
        pltpu.VMEM_SHARED((NP,), jnp.float32),
    ],
    compiler_params=pltpu.CompilerParams(
        needs_layout_passes=False, use_tc_tiling_on_sc=False
    ),
)
def _deg_kernel(dst_hbm, w_hbm, out_hbm, dstb, wb, zbuf, acc):
    c = lax.axis_index("c")
    s = lax.axis_index("s")
    wid = s * NC + c
    zero = jnp.zeros((L,), jnp.float32)

    @pl.loop(0, RPT // L)
    def _(i):
        zbuf[pl.ds(i * L, L)] = zero

    row0 = s * RPT
    pltpu.sync_copy(zbuf, acc.at[pl.ds(row0, RPT)])
    plsc.subcore_barrier()

    @pl.loop(0, NSB)
    def _(b):
        pltpu.sync_copy(dst_hbm.at[wid, pl.ds(b * SB, SB)], dstb)
        pltpu.sync_copy(w_hbm.at[wid, pl.ds(b * SB, SB)], wb)

        @pl.loop(0, SB)
        def _(j):
            pltpu.sync_copy(wb.at[j], acc.at[dstb.at[j]], add=True)

    plsc.subcore_barrier()
    pltpu.sync_copy(acc.at[pl.ds(row0, RPT)], out_hbm.at[c, pl.ds(row0, RPT)])


def _make_agg(D, tiled=True):
    @functools.partial(
        pl.kernel,
        out_type=jax.ShapeDtypeStruct((NC, NP, D), jnp.float32),
        mesh=_MESH,
        scratch_types=[
            pltpu.VMEM((SB, C), jnp.int32),
            pltpu.VMEM((SB, C), jnp.int32),
            pltpu.VMEM((SB, C), jnp.float32),
            pltpu.VMEM((C, D), jnp.float32),
            pltpu.VMEM((C, D), jnp.float32),
            pltpu.VMEM((ZR, D), jnp.float32),
            pltpu.VMEM_SHARED((NP, D), jnp.float32),
            pltpu.SemaphoreType.DMA,
            pltpu.SemaphoreType.DMA,
            pltpu.SemaphoreType.DMA,
            pltpu.SemaphoreType.DMA,
        ],
        compiler_params=(
            _SC_PARAMS
            if tiled
            else pltpu.CompilerParams(
                needs_layout_passes=False, use_tc_tiling_on_sc=False
            )
        ),
    )
    def _agg(h_hbm, src_hbm, dst_hbm, w_hbm, out_hbm,
             srcb, dstb, wb, rows0, rows1, zbuf, acc,
             gsem0, gsem1, ssem0, ssem1):
        c = lax.axis_index("c")
        s = lax.axis_index("s")
        wid = s * NC + c

        zero = jnp.zeros((L,), jnp.float32)

        @pl.loop(0, ZR)
        def _(i):
            for d in range(D // L):
                zbuf[i, pl.ds(d * L, L)] = zero

        row0 = s * RPT
        for k in range(RPT // ZR):
            pltpu.sync_copy(zbuf, acc.at[pl.ds(row0 + k * ZR, ZR)])
        plsc.subcore_barrier()

        rbufs = (rows0, rows1)
        gsems = (gsem0, gsem1)
        ssems = (ssem0, ssem1)

        @pl.loop(0, NSB)
        def _(b):
            pltpu.sync_copy(src_hbm.at[wid, pl.ds(b * SB, SB)], srcb)
            pltpu.sync_copy(dst_hbm.at[wid, pl.ds(b * SB, SB)], dstb)
            pltpu.sync_copy(w_hbm.at[wid, pl.ds(b * SB, SB)], wb)

            gd = {0: pltpu.async_copy(h_hbm.at[srcb.at[0]], rows0, gsem0)}
            sd = {}
            for jj in range(SB):
                p = jj % 2
                rb = rbufs[p]
                gd[jj].wait()
                if jj + 1 < SB:
                    if jj >= 1:
                        sd[jj - 1].wait()
                    q = (jj + 1) % 2
                    gd[jj + 1] = pltpu.async_copy(
                        h_hbm.at[srcb.at[jj + 1]], rbufs[q], gsems[q]
                    )

                @plsc.parallel_loop(0, C, unroll=5)
                def _(i, jj=jj, rb=rb):
                    j16 = jnp.full((L,), jj, jnp.int32)
                    i16 = jnp.full((L,), i, jnp.int32)
                    wv = plsc.load_gather(wb, [j16, i16])
                    for d in range(D // L):
                        rb[i, pl.ds(d * L, L)] = rb[i, pl.ds(d * L, L)] * wv

                sd[jj] = pltpu.async_copy(
                    rb, acc.at[dstb.at[jj]], ssems[p], add=True
                )
            sd[SB - 2].wait()
            sd[SB - 1].wait()

        plsc.subcore_barrier()
        pltpu.sync_copy(acc.at[pl.ds(row0, RPT)],
                        out_hbm.at[c, pl.ds(row0, RPT)])

    return _agg


_agg128 = _make_agg(128)
_agg64 = _make_agg(64, tiled=False)


_BR = 1000


def _prep_body(x_ref, degp_ref, wc_ref, ws_ref, bs_ref,
               h1s_ref, hls_ref, hsp_ref, dinv_ref):
    xb = x_ref[...]
    deg = 1.0 + jnp.sum(degp_ref[...], axis=1)
    dinv = jnp.where(deg > 0, lax.rsqrt(deg), 0.0)
    h = jnp.dot(xb, wc_ref[...], preferred_element_type=jnp.float32)
    h = h * dinv[:, None]
    h1s_ref[...] = h[:, :128]
    hls_ref[...] = h[:, 128:192]
    hsp_ref[...] = (
        jnp.dot(xb, ws_ref[...], preferred_element_type=jnp.float32)
        + bs_ref[...]
    )
    dinv_ref[...] = dinv[:, None]


def _mid_body(p1_ref, pl_ref, h1s_ref, hls_ref, dinv_ref, hsp_ref,
              b1_ref, bl1_ref, wl2_ref, bl2_ref, w2_ref,
              h2s_ref, leader_ref):
    dinv = dinv_ref[...]
    t1 = (p1_ref[0] + p1_ref[1] + h1s_ref[...]) * dinv
    tl = (pl_ref[0] + pl_ref[1] + hls_ref[...]) * dinv
    hn = jnp.maximum(t1 + b1_ref[...], 0.0)
    hl = jnp.maximum(tl + bl1_ref[...], 0.0)
    logit = jnp.sum(hl * wl2_ref[...], axis=1, keepdims=True) + bl2_ref[...]
    leader = jax.nn.sigmoid(logit)
    hnew = (1.0 - leader) * hn + leader * hsp_ref[...]
    h2s_ref[...] = (
        jnp.dot(hnew, w2_ref[...], preferred_element_type=jnp.float32) * dinv
    )
    leader_ref[...] = leader


def _fin_body(q_ref, h2s_ref, dinv_ref, b2_ref, out_ref):
    t = (q_ref[0] + q_ref[1] + h2s_ref[...]) * dinv_ref[...]
    out_ref[...] = jnp.maximum(t + b2_ref[...], 0.0)


def _row_spec(shape):
    if len(shape) == 2:
        return pl.BlockSpec((_BR, shape[1]), lambda i: (i, 0))
    return pl.BlockSpec((shape[0], _BR, shape[2]), lambda i: (0, i, 0))


def _full_spec(shape):
    return pl.BlockSpec(shape, lambda i: tuple(0 for _ in shape))


def kernel(x, edge_index, edge_weight, W1, b1, W2, b2, Ws, bs,
           Wl1, bl1, Wl2, bl2):
    src = edge_index[0].reshape(NW, CPT, C)
    dst = edge_index[1].reshape(NW, CPT, C)
    w = edge_weight.reshape(NW, CPT, C)

    degp = _deg_kernel(dst, w)[:, :N].T

    Wc = jnp.concatenate([W1, Wl1], axis=1)
    grid = (N // _BR,)
    h1s, hls, hsp, dinv = pl.pallas_call(
        _prep_body,
        grid=grid,
        in_specs=[
            _row_spec((N, 128)),
            pl.BlockSpec((_BR, NC), lambda i: (i, 0)),
            _full_spec((128, 192)),
            _full_spec((128, 128)),
            _full_spec((1, 128)),
        ],
        out_specs=[
            _row_spec((N, 128)),
            _row_spec((N, 64)),
            _row_spec((N, 128)),
            _row_spec((N, 1)),
        ],
        out_shape=[
            jax.ShapeDtypeStruct((N, 128), jnp.float32),
            jax.ShapeDtypeStruct((N, 64), jnp.float32),
            jax.ShapeDtypeStruct((N, 128), jnp.float32),
            jax.ShapeDtypeStruct((N, 1), jnp.float32),
        ],
    )(x, degp, Wc, Ws, bs.reshape(1, 128))

    p1 = _agg128(h1s, src, dst, w)
    plp = _agg64(hls, src, dst, w)

    h2s, leader = pl.pallas_call(
        _mid_body,
        grid=grid,
        in_specs=[
            _row_spec((NC, NP, 128)),
            _row_spec((NC, NP, 64)),
            _row_spec((N, 128)),
            _row_spec((N, 64)),
            _row_spec((N, 1)),
            _row_spec((N, 128)),
            _full_spec((1, 128)),
            _full_spec((1, 64)),
            _full_spec((1, 64)),
            _full_spec((1, 1)),
            _full_spec((128, 128)),
        ],
        out_specs=[
            _row_spec((N, 128)),
            _row_spec((N, 1)),
        ],
        out_shape=[
            jax.ShapeDtypeStruct((N, 128), jnp.float32),
            jax.ShapeDtypeStruct((N, 1), jnp.float32),
        ],
    )(p1, plp, h1s, hls, dinv, hsp, b1.reshape(1, 128), bl1.reshape(1, 64),
      Wl2.reshape(1, 64), bl2.reshape(1, 1), W2)

    q = _agg128(h2s, src, dst, w)

    h_final = pl.pallas_call(
        _fin_body,
        grid=grid,
        in_specs=[
            _row_spec((NC, NP, 128)),
            _row_spec((N, 128)),
            _row_spec((N, 1)),
            _full_spec((1, 128)),
        ],
        out_specs=_row_spec((N, 128)),
        out_shape=jax.ShapeDtypeStruct((N, 128), jnp.float32),
    )(q, h2s, dinv, b2.reshape(1, 128))

    return (h_final, leader)


# speedup vs baseline: 1.1477x; 1.0028x over previous
import functools

import jax
import jax.numpy as jnp
from jax import lax
from jax.experimental import pallas as pl
from jax.experimental.pallas import tpu as pltpu
from jax.experimental.pallas import tpu_sc as plsc

N = 10000
NP = 10240
E = 320000
NC, NS, L = 2, 16, 16
NW = NC * NS
C = 125
CPT = E // (NW * C)
SB = 40
NSB = CPT // SB
RPT = NP // NS
ZR = 8

_MESH = plsc.VectorSubcoreMesh(
    core_axis_name="c", subcore_axis_name="s", num_cores=NC, num_subcores=NS
)
_SC_PARAMS = pltpu.CompilerParams(needs_layout_passes=False)


@functools.partial(
    pl.kernel,
    out_type=jax.ShapeDtypeStruct((NC, NP), jnp.float32),
    mesh=_MESH,
    scratch_types=[
        pltpu.VMEM((SB, C), jnp.int32),
        pltpu.VMEM((SB, C), jnp.float32),
        pltpu.VMEM((RPT,), jnp.float32),
        pltpu.VMEM_SHARED((NP,), jnp.float32),
    ],
    compiler_params=pltpu.CompilerParams(
        needs_layout_passes=False, use_tc_tiling_on_sc=False
    ),
)
def _deg_kernel(dst_hbm, w_hbm, out_hbm, dstb, wb, zbuf, acc):
    c = lax.axis_index("c")
    s = lax.axis_index("s")
    wid = s * NC + c
    zero = jnp.zeros((L,), jnp.float32)

    @pl.loop(0, RPT // L)
    def _(i):
        zbuf[pl.ds(i * L, L)] = zero

    row0 = s * RPT
    pltpu.sync_copy(zbuf, acc.at[pl.ds(row0, RPT)])
    plsc.subcore_barrier()

    @pl.loop(0, NSB)
    def _(b):
        pltpu.sync_copy(dst_hbm.at[wid, pl.ds(b * SB, SB)], dstb)
        pltpu.sync_copy(w_hbm.at[wid, pl.ds(b * SB, SB)], wb)

        @pl.loop(0, SB)
        def _(j):
            pltpu.sync_copy(wb.at[j], acc.at[dstb.at[j]], add=True)

    plsc.subcore_barrier()
    pltpu.sync_copy(acc.at[pl.ds(row0, RPT)], out_hbm.at[c, pl.ds(row0, RPT)])


def _make_agg(D, tiled=True):
    @functools.partial(
        pl.kernel,
        out_type=jax.ShapeDtypeStruct((NC, NP, D), jnp.float32),
        mesh=_MESH,
        scratch_types=[
            pltpu.VMEM((SB, C), jnp.int32),
            pltpu.VMEM((SB, C), jnp.int32),
            pltpu.VMEM((SB, C), jnp.float32),
            pltpu.VMEM((C, D), jnp.float32),
            pltpu.VMEM((C, D), jnp.float32),
            pltpu.VMEM((ZR, D), jnp.float32),
            pltpu.VMEM_SHARED((NP, D), jnp.float32),
            pltpu.SemaphoreType.DMA,
            pltpu.SemaphoreType.DMA,
            pltpu.SemaphoreType.DMA,
            pltpu.SemaphoreType.DMA,
        ],
        compiler_params=(
            _SC_PARAMS
            if tiled
            else pltpu.CompilerParams(
                needs_layout_passes=False, use_tc_tiling_on_sc=False
            )
        ),
    )
    def _agg(h_hbm, src_hbm, dst_hbm, w_hbm, out_hbm,
             srcb, dstb, wb, rows0, rows1, zbuf, acc,
             gsem0, gsem1, ssem0, ssem1):
        c = lax.axis_index("c")
        s = lax.axis_index("s")
        wid = s * NC + c

        zero = jnp.zeros((L,), jnp.float32)

        @pl.loop(0, ZR)
        def _(i):
            for d in range(D // L):
                zbuf[i, pl.ds(d * L, L)] = zero

        row0 = s * RPT
        for k in range(RPT // ZR):
            pltpu.sync_copy(zbuf, acc.at[pl.ds(row0 + k * ZR, ZR)])
        plsc.subcore_barrier()

        rbufs = (rows0, rows1)
        gsems = (gsem0, gsem1)
        ssems = (ssem0, ssem1)

        @pl.loop(0, NSB)
        def _(b):
            pltpu.sync_copy(src_hbm.at[wid, pl.ds(b * SB, SB)], srcb)
            pltpu.sync_copy(dst_hbm.at[wid, pl.ds(b * SB, SB)], dstb)
            pltpu.sync_copy(w_hbm.at[wid, pl.ds(b * SB, SB)], wb)

            gd = {0: pltpu.async_copy(h_hbm.at[srcb.at[0]], rows0, gsem0)}
            sd = {}
            for jj in range(SB):
                p = jj % 2
                rb = rbufs[p]
                gd[jj].wait()
                if jj + 1 < SB:
                    if jj >= 1:
                        sd[jj - 1].wait()
                    q = (jj + 1) % 2
                    gd[jj + 1] = pltpu.async_copy(
                        h_hbm.at[srcb.at[jj + 1]], rbufs[q], gsems[q]
                    )

                @plsc.parallel_loop(0, C, unroll=4)
                def _(i, jj=jj, rb=rb):
                    j16 = jnp.full((L,), jj, jnp.int32)
                    i16 = jnp.full((L,), i, jnp.int32)
                    wv = plsc.load_gather(wb, [j16, i16])
                    for d in range(D // L):
                        rb[i, pl.ds(d * L, L)] = rb[i, pl.ds(d * L, L)] * wv

                sd[jj] = pltpu.async_copy(
                    rb, acc.at[dstb.at[jj]], ssems[p], add=True
                )
            sd[SB - 2].wait()
            sd[SB - 1].wait()

        plsc.subcore_barrier()
        pltpu.sync_copy(acc.at[pl.ds(row0, RPT)],
                        out_hbm.at[c, pl.ds(row0, RPT)])

    return _agg


_agg128 = _make_agg(128)
_agg64 = _make_agg(64, tiled=False)


_BR = 1000


def _prep_body(x_ref, degp_ref, wc_ref, ws_ref, bs_ref,
               h1s_ref, hls_ref, hsp_ref, dinv_ref):
    xb = x_ref[...]
    deg = 1.0 + jnp.sum(degp_ref[...], axis=1)
    dinv = jnp.where(deg > 0, lax.rsqrt(deg), 0.0)
    h = jnp.dot(xb, wc_ref[...], preferred_element_type=jnp.float32)
    h = h * dinv[:, None]
    h1s_ref[...] = h[:, :128]
    hls_ref[...] = h[:, 128:192]
    hsp_ref[...] = (
        jnp.dot(xb, ws_ref[...], preferred_element_type=jnp.float32)
        + bs_ref[...]
    )
    dinv_ref[...] = dinv[:, None]


def _mid_body(p1_ref, pl_ref, h1s_ref, hls_ref, dinv_ref, hsp_ref,
              b1_ref, bl1_ref, wl2_ref, bl2_ref, w2_ref,
              h2s_ref, leader_ref):
    dinv = dinv_ref[...]
    t1 = (p1_ref[0] + p1_ref[1] + h1s_ref[...]) * dinv
    tl = (pl_ref[0] + pl_ref[1] + hls_ref[...]) * dinv
    hn = jnp.maximum(t1 + b1_ref[...], 0.0)
    hl = jnp.maximum(tl + bl1_ref[...], 0.0)
    logit = jnp.sum(hl * wl2_ref[...], axis=1, keepdims=True) + bl2_ref[...]
    leader = jax.nn.sigmoid(logit)
    hnew = (1.0 - leader) * hn + leader * hsp_ref[...]
    h2s_ref[...] = (
        jnp.dot(hnew, w2_ref[...], preferred_element_type=jnp.float32) * dinv
    )
    leader_ref[...] = leader


def _fin_body(q_ref, h2s_ref, dinv_ref, b2_ref, out_ref):
    t = (q_ref[0] + q_ref[1] + h2s_ref[...]) * dinv_ref[...]
    out_ref[...] = jnp.maximum(t + b2_ref[...], 0.0)


def _row_spec(shape):
    if len(shape) == 2:
        return pl.BlockSpec((_BR, shape[1]), lambda i: (i, 0))
    return pl.BlockSpec((shape[0], _BR, shape[2]), lambda i: (0, i, 0))


def _full_spec(shape):
    return pl.BlockSpec(shape, lambda i: tuple(0 for _ in shape))


def kernel(x, edge_index, edge_weight, W1, b1, W2, b2, Ws, bs,
           Wl1, bl1, Wl2, bl2):
    src = edge_index[0].reshape(NW, CPT, C)
    dst = edge_index[1].reshape(NW, CPT, C)
    w = edge_weight.reshape(NW, CPT, C)

    degp = _deg_kernel(dst, w)[:, :N].T

    Wc = jnp.concatenate([W1, Wl1], axis=1)
    grid = (N // _BR,)
    h1s, hls, hsp, dinv = pl.pallas_call(
        _prep_body,
        grid=grid,
        in_specs=[
            _row_spec((N, 128)),
            pl.BlockSpec((_BR, NC), lambda i: (i, 0)),
            _full_spec((128, 192)),
            _full_spec((128, 128)),
            _full_spec((1, 128)),
        ],
        out_specs=[
            _row_spec((N, 128)),
            _row_spec((N, 64)),
            _row_spec((N, 128)),
            _row_spec((N, 1)),
        ],
        out_shape=[
            jax.ShapeDtypeStruct((N, 128), jnp.float32),
            jax.ShapeDtypeStruct((N, 64), jnp.float32),
            jax.ShapeDtypeStruct((N, 128), jnp.float32),
            jax.ShapeDtypeStruct((N, 1), jnp.float32),
        ],
    )(x, degp, Wc, Ws, bs.reshape(1, 128))

    p1 = _agg128(h1s, src, dst, w)
    plp = _agg64(hls, src, dst, w)

    h2s, leader = pl.pallas_call(
        _mid_body,
        grid=grid,
        in_specs=[
            _row_spec((NC, NP, 128)),
            _row_spec((NC, NP, 64)),
            _row_spec((N, 128)),
            _row_spec((N, 64)),
            _row_spec((N, 1)),
            _row_spec((N, 128)),
            _full_spec((1, 128)),
            _full_spec((1, 64)),
            _full_spec((1, 64)),
            _full_spec((1, 1)),
            _full_spec((128, 128)),
        ],
        out_specs=[
            _row_spec((N, 128)),
            _row_spec((N, 1)),
        ],
        out_shape=[
            jax.ShapeDtypeStruct((N, 128), jnp.float32),
            jax.ShapeDtypeStruct((N, 1), jnp.float32),
        ],
    )(p1, plp, h1s, hls, dinv, hsp, b1.reshape(1, 128), bl1.reshape(1, 64),
      Wl2.reshape(1, 64), bl2.reshape(1, 1), W2)

    q = _agg128(h2s, src, dst, w)

    h_final = pl.pallas_call(
        _fin_body,
        grid=grid,
        in_specs=[
            _row_spec((NC, NP, 128)),
            _row_spec((N, 128)),
            _row_spec((N, 1)),
            _full_spec((1, 128)),
        ],
        out_specs=_row_spec((N, 128)),
        out_shape=jax.ShapeDtypeStruct((N, 128), jnp.float32),
    )(q, h2s, dinv, b2.reshape(1, 128))

    return (h_final, leader)


# speedup vs baseline: 1.1510x; 1.0028x over previous
import functools

import jax
import jax.numpy as jnp
from jax import lax
from jax.experimental import pallas as pl
from jax.experimental.pallas import tpu as pltpu
from jax.experimental.pallas import tpu_sc as plsc

N = 10000
NP = 10240
E = 320000
NC, NS, L = 2, 16, 16
NW = NC * NS
C = 125
CPT = E // (NW * C)
SB = 40
NSB = CPT // SB
RPT = NP // NS
ZR = 8

_MESH = plsc.VectorSubcoreMesh(
    core_axis_name="c", subcore_axis_name="s", num_cores=NC, num_subcores=NS
)
_SC_PARAMS = pltpu.CompilerParams(needs_layout_passes=False)


@functools.partial(
    pl.kernel,
    out_type=jax.ShapeDtypeStruct((NC, NP), jnp.float32),
    mesh=_MESH,
    scratch_types=[
        pltpu.VMEM((SB, C), jnp.int32),
        pltpu.VMEM((SB, C), jnp.float32),
        pltpu.VMEM((RPT,), jnp.float32),
        pltpu.VMEM_SHARED((NP,), jnp.float32),
    ],
    compiler_params=pltpu.CompilerParams(
        needs_layout_passes=False, use_tc_tiling_on_sc=False
    ),
)
def _deg_kernel(dst_hbm, w_hbm, out_hbm, dstb, wb, zbuf, acc):
    c = lax.axis_index("c")
    s = lax.axis_index("s")
    wid = s * NC + c
    zero = jnp.zeros((L,), jnp.float32)

    @pl.loop(0, RPT // L)
    def _(i):
        zbuf[pl.ds(i * L, L)] = zero

    row0 = s * RPT
    pltpu.sync_copy(zbuf, acc.at[pl.ds(row0, RPT)])
    plsc.subcore_barrier()

    @pl.loop(0, NSB)
    def _(b):
        pltpu.sync_copy(dst_hbm.at[wid, pl.ds(b * SB, SB)], dstb)
        pltpu.sync_copy(w_hbm.at[wid, pl.ds(b * SB, SB)], wb)

        @pl.loop(0, SB)
        def _(j):
            pltpu.sync_copy(wb.at[j], acc.at[dstb.at[j]], add=True)

    plsc.subcore_barrier()
    pltpu.sync_copy(acc.at[pl.ds(row0, RPT)], out_hbm.at[c, pl.ds(row0, RPT)])


def _make_agg(D, tiled=True):
    @functools.partial(
        pl.kernel,
        out_type=jax.ShapeDtypeStruct((NC, NP, D), jnp.float32),
        mesh=_MESH,
        scratch_types=[
            pltpu.VMEM((SB, C), jnp.int32),
            pltpu.VMEM((SB, C), jnp.int32),
            pltpu.VMEM((SB, C), jnp.float32),
            pltpu.VMEM((C, D), jnp.float32),
            pltpu.VMEM((C, D), jnp.float32),
            pltpu.VMEM((ZR, D), jnp.float32),
            pltpu.VMEM_SHARED((NP, D), jnp.float32),
            pltpu.SemaphoreType.DMA,
            pltpu.SemaphoreType.DMA,
            pltpu.SemaphoreType.DMA,
            pltpu.SemaphoreType.DMA,
        ],
        compiler_params=(
            _SC_PARAMS
            if tiled
            else pltpu.CompilerParams(
                needs_layout_passes=False, use_tc_tiling_on_sc=False
            )
        ),
    )
    def _agg(h_hbm, src_hbm, dst_hbm, w_hbm, out_hbm,
             srcb, dstb, wb, rows0, rows1, zbuf, acc,
             gsem0, gsem1, ssem0, ssem1):
        c = lax.axis_index("c")
        s = lax.axis_index("s")
        wid = s * NC + c

        zero = jnp.zeros((L,), jnp.float32)

        @pl.loop(0, ZR)
        def _(i):
            for d in range(D // L):
                zbuf[i, pl.ds(d * L, L)] = zero

        row0 = s * RPT
        for k in range(RPT // ZR):
            pltpu.sync_copy(zbuf, acc.at[pl.ds(row0 + k * ZR, ZR)])
        plsc.subcore_barrier()

        rbufs = (rows0, rows1)
        gsems = (gsem0, gsem1)
        ssems = (ssem0, ssem1)

        @pl.loop(0, NSB)
        def _(b):
            pltpu.sync_copy(src_hbm.at[wid, pl.ds(b * SB, SB)], srcb)
            pltpu.sync_copy(dst_hbm.at[wid, pl.ds(b * SB, SB)], dstb)
            pltpu.sync_copy(w_hbm.at[wid, pl.ds(b * SB, SB)], wb)

            gd = {0: pltpu.async_copy(h_hbm.at[srcb.at[0]], rows0, gsem0)}
            sd = {}
            for jj in range(SB):
                p = jj % 2
                rb = rbufs[p]
                gd[jj].wait()
                if jj + 1 < SB:
                    if jj >= 1:
                        sd[jj - 1].wait()
                    q = (jj + 1) % 2
                    gd[jj + 1] = pltpu.async_copy(
                        h_hbm.at[srcb.at[jj + 1]], rbufs[q], gsems[q]
                    )

                @plsc.parallel_loop(0, C, unroll=5)
                def _(i, jj=jj, rb=rb):
                    j16 = jnp.full((L,), jj, jnp.int32)
                    i16 = jnp.full((L,), i, jnp.int32)
                    wv = plsc.load_gather(wb, [j16, i16])
                    for d in range(D // L):
                        rb[i, pl.ds(d * L, L)] = rb[i, pl.ds(d * L, L)] * wv

                sd[jj] = pltpu.async_copy(
                    rb, acc.at[dstb.at[jj]], ssems[p], add=True
                )
            sd[SB - 2].wait()
            sd[SB - 1].wait()

        plsc.subcore_barrier()
        pltpu.sync_copy(acc.at[pl.ds(row0, RPT)],
                        out_hbm.at[c, pl.ds(row0, RPT)])

    return _agg


_agg128 = _make_agg(128)
_agg64 = _make_agg(64, tiled=False)


_BR = 1000


def _prep_body(x_ref, degp_ref, wc_ref, ws_ref, bs_ref,
               h1s_ref, hls_ref, hsp_ref, dinv_ref):
    xb = x_ref[...]
    deg = 1.0 + jnp.sum(degp_ref[...], axis=1)
    dinv = jnp.where(deg > 0, lax.rsqrt(deg), 0.0)
    h = jnp.dot(xb, wc_ref[...], preferred_element_type=jnp.float32)
    h = h * dinv[:, None]
    h1s_ref[...] = h[:, :128]
    hls_ref[...] = h[:, 128:192]
    hsp_ref[...] = (
        jnp.dot(xb, ws_ref[...], preferred_element_type=jnp.float32)
        + bs_ref[...]
    )
    dinv_ref[...] = dinv[:, None]


def _mid_body(p1_ref, pl_ref, h1s_ref, hls_ref, dinv_ref, hsp_ref,
              b1_ref, bl1_ref, wl2_ref, bl2_ref, w2_ref,
              h2s_ref, leader_ref):
    dinv = dinv_ref[...]
    t1 = (p1_ref[0] + p1_ref[1] + h1s_ref[...]) * dinv
    tl = (pl_ref[0] + pl_ref[1] + hls_ref[...]) * dinv
    hn = jnp.maximum(t1 + b1_ref[...], 0.0)
    hl = jnp.maximum(tl + bl1_ref[...], 0.0)
    logit = jnp.sum(hl * wl2_ref[...], axis=1, keepdims=True) + bl2_ref[...]
    leader = jax.nn.sigmoid(logit)
    hnew = (1.0 - leader) * hn + leader * hsp_ref[...]
    h2s_ref[...] = (
        jnp.dot(hnew, w2_ref[...], preferred_element_type=jnp.float32) * dinv
    )
    leader_ref[...] = leader


def _fin_body(q_ref, h2s_ref, dinv_ref, b2_ref, out_ref):
    t = (q_ref[0] + q_ref[1] + h2s_ref[...]) * dinv_ref[...]
    out_ref[...] = jnp.maximum(t + b2_ref[...], 0.0)


def _row_spec(shape):
    if len(shape) == 2:
        return pl.BlockSpec((_BR, shape[1]), lambda i: (i, 0))
    return pl.BlockSpec((shape[0], _BR, shape[2]), lambda i: (0, i, 0))


def _full_spec(shape):
    return pl.BlockSpec(shape, lambda i: tuple(0 for _ in shape))


def kernel(x, edge_index, edge_weight, W1, b1, W2, b2, Ws, bs,
           Wl1, bl1, Wl2, bl2):
    src = edge_index[0].reshape(NW, CPT, C)
    dst = edge_index[1].reshape(NW, CPT, C)
    w = edge_weight.reshape(NW, CPT, C)

    degp = _deg_kernel(dst, w)[:, :N].T

    Wc = jnp.concatenate([W1, Wl1], axis=1)
    grid = (N // _BR,)
    h1s, hls, hsp, dinv = pl.pallas_call(
        _prep_body,
        grid=grid,
        in_specs=[
            _row_spec((N, 128)),
            pl.BlockSpec((_BR, NC), lambda i: (i, 0)),
            _full_spec((128, 192)),
            _full_spec((128, 128)),
            _full_spec((1, 128)),
        ],
        out_specs=[
            _row_spec((N, 128)),
            _row_spec((N, 64)),
            _row_spec((N, 128)),
            _row_spec((N, 1)),
        ],
        out_shape=[
            jax.ShapeDtypeStruct((N, 128), jnp.float32),
            jax.ShapeDtypeStruct((N, 64), jnp.float32),
            jax.ShapeDtypeStruct((N, 128), jnp.float32),
            jax.ShapeDtypeStruct((N, 1), jnp.float32),
        ],
    )(x, degp, Wc, Ws, bs.reshape(1, 128))

    p1 = _agg128(h1s, src, dst, w)
    plp = _agg64(hls, src, dst, w)

    h2s, leader = pl.pallas_call(
        _mid_body,
        grid=grid,
        in_specs=[
            _row_spec((NC, NP, 128)),
            _row_spec((NC, NP, 64)),
            _row_spec((N, 128)),
            _row_spec((N, 64)),
            _row_spec((N, 1)),
            _row_spec((N, 128)),
            _full_spec((1, 128)),
            _full_spec((1, 64)),
            _full_spec((1, 64)),
            _full_spec((1, 1)),
            _full_spec((128, 128)),
        ],
        out_specs=[
            _row_spec((N, 128)),
            _row_spec((N, 1)),
        ],
        out_shape=[
            jax.ShapeDtypeStruct((N, 128), jnp.float32),
            jax.ShapeDtypeStruct((N, 1), jnp.float32),
        ],
    )(p1, plp, h1s, hls, dinv, hsp, b1.reshape(1, 128), bl1.reshape(1, 64),
      Wl2.reshape(1, 64), bl2.reshape(1, 1), W2)

    q = _agg128(h2s, src, dst, w)

    h_final = pl.pallas_call(
        _fin_body,
        grid=grid,
        in_specs=[
            _row_spec((NC, NP, 128)),
            _row_spec((N, 128)),
            _row_spec((N, 1)),
            _full_spec((1, 128)),
        ],
        out_specs=_row_spec((N, 128)),
        out_shape=jax.ShapeDtypeStruct((N, 128), jnp.float32),
    )(q, h2s, dinv, b2.reshape(1, 128))

    return (h_final, leader)
